# baseline (speedup 1.0000x reference)
import functools

import jax
import jax.numpy as jnp
from jax import lax
from jax.experimental import pallas as pl
from jax.experimental.pallas import tpu as pltpu
from jax.experimental.pallas import tpu_sc as plsc

TILE = 128


def _gate_body(x_ref, gw_ref, alpha_ref, eid_ref, w_ref, *, n_experts):
    x = x_ref[...]
    logits = jnp.dot(x, gw_ref[...], preferred_element_type=jnp.float32)
    t, lanes = logits.shape
    col = lax.broadcasted_iota(jnp.int32, (t, lanes), 1)
    in_cols = col < n_experts
    logits = jnp.where(in_cols, logits, -1e30)
    lmax = jnp.max(logits, axis=1, keepdims=True)
    ssum = jnp.sum(jnp.where(in_cols, jnp.exp(logits - lmax), 0.0), axis=1,
                   keepdims=True)
    eidx = jnp.min(jnp.where(logits == lmax, col, n_experts), axis=1)
    alpha_sel = jnp.sum(
        jnp.where(col == eidx[:, None], alpha_ref[...], 0.0), axis=1)
    eid_ref[...] = eidx
    w_ref[...] = alpha_sel / ssum[:, 0]


def _gate(x, gate_w, alpha):
    t, h = x.shape
    e = gate_w.shape[1]
    gwp = jnp.pad(gate_w, ((0, 0), (0, 128 - e)))
    alphap = jnp.pad(alpha, (0, 128 - e)).reshape(1, 128)
    return pl.pallas_call(
        functools.partial(_gate_body, n_experts=e),
        out_shape=(jax.ShapeDtypeStruct((t,), jnp.int32),
                   jax.ShapeDtypeStruct((t,), jnp.float32)),
    )(x, gwp, alphap)


def _sc_gather_rows(table, idx):
    t, h = table.shape
    info = plsc.get_sparse_core_info()
    nw = info.num_cores * info.num_subcores
    b_per_w = t // nw
    mesh = plsc.VectorSubcoreMesh(core_axis_name="c", subcore_axis_name="s")

    @functools.partial(
        pl.kernel, mesh=mesh,
        out_type=jax.ShapeDtypeStruct((t, h), table.dtype),
        scratch_types=[
            pltpu.VMEM((b_per_w,), jnp.int32),
            pltpu.VMEM((b_per_w, h), table.dtype),
            pltpu.SemaphoreType.DMA,
        ],
    )
    def k(table_hbm, idx_hbm, out_hbm, idx_v, rows_v, sem):
        wid = lax.axis_index("s") * info.num_cores + lax.axis_index("c")
        base = wid * b_per_w
        pltpu.sync_copy(idx_hbm.at[pl.ds(base, b_per_w)], idx_v)
        pltpu.async_copy(table_hbm.at[idx_v], rows_v, sem).wait()
        pltpu.sync_copy(rows_v, out_hbm.at[pl.ds(base, b_per_w)])

    return k(table, idx)


def _sc_dispatch(x, wpad, rank):
    t, h = x.shape
    hw = wpad.shape[1]
    info = plsc.get_sparse_core_info()
    nw = info.num_cores * info.num_subcores
    b_per_w = t // nw
    mesh = plsc.VectorSubcoreMesh(core_axis_name="c", subcore_axis_name="s")

    @functools.partial(
        pl.kernel, mesh=mesh,
        out_type=(jax.ShapeDtypeStruct((t, h), x.dtype),
                  jax.ShapeDtypeStruct((t, hw), wpad.dtype)),
        scratch_types=[
            pltpu.VMEM((b_per_w,), jnp.int32),
            pltpu.VMEM((b_per_w, h), x.dtype),
            pltpu.VMEM((b_per_w, hw), wpad.dtype),
            pltpu.SemaphoreType.DMA,
            pltpu.SemaphoreType.DMA,
        ],
    )
    def k(x_hbm, w_hbm, rank_hbm, xs_hbm, ws_hbm, idx_v, xrows_v, wrows_v,
          sem1, sem2):
        wid = lax.axis_index("s") * info.num_cores + lax.axis_index("c")
        base = wid * b_per_w
        pltpu.sync_copy(rank_hbm.at[pl.ds(base, b_per_w)], idx_v)
        pltpu.sync_copy(x_hbm.at[pl.ds(base, b_per_w)], xrows_v)
        pltpu.sync_copy(w_hbm.at[pl.ds(base, b_per_w)], wrows_v)
        cx = pltpu.async_copy(xrows_v, xs_hbm.at[idx_v], sem1)
        cw = pltpu.async_copy(wrows_v, ws_hbm.at[idx_v], sem2)
        cx.wait()
        cw.wait()

    return k(x, wpad, rank)


def _moe_body(meta_ref, x_ref, w1a_ref, w1b_ref, b1_ref, w2a_ref, w2b_ref,
              b2_ref, wrow_ref, out_ref):
    e = pl.program_id(0)
    tile_lo = meta_ref[0, e]
    tile_hi = meta_ref[1, e]
    seg_lo = meta_ref[2, e]
    seg_hi = meta_ref[3, e]
    hh = w1a_ref.shape[1]
    fh = w2a_ref.shape[1]

    @pl.when(e == 0)
    def _():
        out_ref[...] = jnp.zeros_like(out_ref)

    def tile_step(j, _):
        rows = pl.ds(j * TILE, TILE)
        x = x_ref[rows, :].astype(jnp.bfloat16)
        h = jnp.dot(x[:, :hh], w1a_ref[0].astype(jnp.bfloat16),
                    preferred_element_type=jnp.float32)
        h += jnp.dot(x[:, hh:], w1b_ref[0].astype(jnp.bfloat16),
                     preferred_element_type=jnp.float32)
        h = jax.nn.gelu(h + b1_ref[0]).astype(jnp.bfloat16)
        y = jnp.dot(h[:, :fh], w2a_ref[0].astype(jnp.bfloat16),
                    preferred_element_type=jnp.float32)
        y += jnp.dot(h[:, fh:], w2b_ref[0].astype(jnp.bfloat16),
                     preferred_element_type=jnp.float32)
        y = y + b2_ref[0]
        gidx = j * TILE + lax.broadcasted_iota(jnp.int32, (TILE, 1), 0)
        coef = jnp.where((gidx >= seg_lo) & (gidx < seg_hi),
                         wrow_ref[rows, 0:1], 0.0)
        out_ref[rows, :] += y * coef
        return 0

    lax.fori_loop(tile_lo, tile_hi + 1, tile_step, 0)


def _grouped_mlp(meta, x_sorted, W1, b1, W2, b2, w_sorted128):
    t, h = x_sorted.shape
    e, _, f = W1.shape
    hw = w_sorted128.shape[1]
    hh = h // 2
    fh = f // 2

    in_specs = [
        pl.BlockSpec((t, h), lambda ei, m: (0, 0)),
        pl.BlockSpec((1, hh, f), lambda ei, m: (ei, 0, 0)),
        pl.BlockSpec((1, hh, f), lambda ei, m: (ei, 1, 0)),
        pl.BlockSpec((1, 1, f), lambda ei, m: (ei, 0, 0)),
        pl.BlockSpec((1, fh, h), lambda ei, m: (ei, 0, 0)),
        pl.BlockSpec((1, fh, h), lambda ei, m: (ei, 1, 0)),
        pl.BlockSpec((1, 1, h), lambda ei, m: (ei, 0, 0)),
        pl.BlockSpec((t, hw), lambda ei, m: (0, 0)),
    ]
    grid_spec = pltpu.PrefetchScalarGridSpec(
        num_scalar_prefetch=1,
        grid=(e,),
        in_specs=in_specs,
        out_specs=pl.BlockSpec((t, h), lambda ei, m: (0, 0)),
    )
    return pl.pallas_call(
        _moe_body,
        grid_spec=grid_spec,
        out_shape=jax.ShapeDtypeStruct((t, h), jnp.float32),
    )(meta, x_sorted, W1, W1, b1.reshape(e, 1, f), W2, W2,
      b2.reshape(e, 1, h), w_sorted128)


def _routing(eid, n_experts):
    t = eid.shape[0]
    i32 = jnp.int32
    oh = (eid[:, None] == jnp.arange(n_experts, dtype=i32)[None, :])
    oh = oh.astype(i32)
    counts = oh.sum(axis=0)
    offsets = jnp.concatenate(
        [jnp.zeros((1,), i32), jnp.cumsum(counts)[:-1].astype(i32)])
    csum = jnp.cumsum(oh, axis=0) - oh
    rank = ((offsets[None, :] + csum) * oh).sum(axis=1).astype(i32)

    seg_lo = offsets
    seg_hi = offsets + counts
    tile_lo = jnp.where(counts > 0, offsets // TILE, 1).astype(i32)
    tile_hi = jnp.where(counts > 0, (seg_hi - 1) // TILE, 0)
    meta = jnp.stack(
        [tile_lo, tile_hi.astype(i32), seg_lo.astype(i32),
         seg_hi.astype(i32)])
    return meta, rank


def kernel(hidden_states, gate_w, alpha, W1, b1, W2, b2):
    t = hidden_states.shape[0]
    e = gate_w.shape[1]

    eid, wcomb = _gate(hidden_states, gate_w, alpha)
    meta, rank = _routing(eid, e)
    wpad = jnp.zeros((t, 128), jnp.float32).at[:, 0].set(wcomb)
    x_sorted, w_sorted128 = _sc_dispatch(hidden_states, wpad, rank)
    y_sorted = _grouped_mlp(meta, x_sorted, W1, b1, W2, b2, w_sorted128)
    return _sc_gather_rows(y_sorted, rank)


# speedup vs baseline: 1.0456x; 1.0456x over previous
import functools

import jax
import jax.numpy as jnp
from jax import lax
from jax.experimental import pallas as pl
from jax.experimental.pallas import tpu as pltpu
from jax.experimental.pallas import tpu_sc as plsc

TILE = 128


def _gate_body(x_ref, gw_ref, alpha_ref, eid_ref, w_ref, *, n_experts):
    x = x_ref[...]
    logits = jnp.dot(x, gw_ref[...], preferred_element_type=jnp.float32)
    t, lanes = logits.shape
    col = lax.broadcasted_iota(jnp.int32, (t, lanes), 1)
    in_cols = col < n_experts
    logits = jnp.where(in_cols, logits, -1e30)
    lmax = jnp.max(logits, axis=1, keepdims=True)
    ssum = jnp.sum(jnp.where(in_cols, jnp.exp(logits - lmax), 0.0), axis=1,
                   keepdims=True)
    eidx = jnp.min(jnp.where(logits == lmax, col, n_experts), axis=1)
    alpha_sel = jnp.sum(
        jnp.where(col == eidx[:, None], alpha_ref[...], 0.0), axis=1)
    eid_ref[...] = eidx
    w_ref[...] = alpha_sel / ssum[:, 0]


def _gate(x, gate_w, alpha):
    t, h = x.shape
    e = gate_w.shape[1]
    gwp = jnp.pad(gate_w, ((0, 0), (0, 128 - e)))
    alphap = jnp.pad(alpha, (0, 128 - e)).reshape(1, 128)
    return pl.pallas_call(
        functools.partial(_gate_body, n_experts=e),
        out_shape=(jax.ShapeDtypeStruct((t,), jnp.int32),
                   jax.ShapeDtypeStruct((t,), jnp.float32)),
    )(x, gwp, alphap)


def _sc_gather_rows(table, idx):
    t, h = table.shape
    info = plsc.get_sparse_core_info()
    nw = info.num_cores * info.num_subcores
    b_per_w = t // nw
    mesh = plsc.VectorSubcoreMesh(core_axis_name="c", subcore_axis_name="s")

    @functools.partial(
        pl.kernel, mesh=mesh,
        out_type=jax.ShapeDtypeStruct((t, h), table.dtype),
        scratch_types=[
            pltpu.VMEM((b_per_w,), jnp.int32),
            pltpu.VMEM((b_per_w, h), table.dtype),
            pltpu.SemaphoreType.DMA,
        ],
    )
    def k(table_hbm, idx_hbm, out_hbm, idx_v, rows_v, sem):
        wid = lax.axis_index("s") * info.num_cores + lax.axis_index("c")
        base = wid * b_per_w
        pltpu.sync_copy(idx_hbm.at[pl.ds(base, b_per_w)], idx_v)
        pltpu.async_copy(table_hbm.at[idx_v], rows_v, sem).wait()
        pltpu.sync_copy(rows_v, out_hbm.at[pl.ds(base, b_per_w)])

    return k(table, idx)


def _sc_dispatch(x, wpad, rank):
    t, h = x.shape
    hw = wpad.shape[1]
    info = plsc.get_sparse_core_info()
    nw = info.num_cores * info.num_subcores
    b_per_w = t // nw
    mesh = plsc.VectorSubcoreMesh(core_axis_name="c", subcore_axis_name="s")

    @functools.partial(
        pl.kernel, mesh=mesh,
        out_type=(jax.ShapeDtypeStruct((t, h), x.dtype),
                  jax.ShapeDtypeStruct((t, hw), wpad.dtype)),
        scratch_types=[
            pltpu.VMEM((b_per_w,), jnp.int32),
            pltpu.VMEM((b_per_w, h), x.dtype),
            pltpu.VMEM((b_per_w, hw), wpad.dtype),
            pltpu.SemaphoreType.DMA,
            pltpu.SemaphoreType.DMA,
        ],
    )
    def k(x_hbm, w_hbm, rank_hbm, xs_hbm, ws_hbm, idx_v, xrows_v, wrows_v,
          sem1, sem2):
        wid = lax.axis_index("s") * info.num_cores + lax.axis_index("c")
        base = wid * b_per_w
        pltpu.sync_copy(rank_hbm.at[pl.ds(base, b_per_w)], idx_v)
        pltpu.sync_copy(x_hbm.at[pl.ds(base, b_per_w)], xrows_v)
        pltpu.sync_copy(w_hbm.at[pl.ds(base, b_per_w)], wrows_v)
        cx = pltpu.async_copy(xrows_v, xs_hbm.at[idx_v], sem1)
        cw = pltpu.async_copy(wrows_v, ws_hbm.at[idx_v], sem2)
        cx.wait()
        cw.wait()

    return k(x, wpad, rank)


NF = 2
STREAMS = 2


def _moe_body(meta_ref, x_ref, *rest):
    w1_refs = rest[:STREAMS]
    b1_ref = rest[STREAMS]
    w2_refs = rest[STREAMS + 1:2 * STREAMS + 1]
    b2_ref = rest[2 * STREAMS + 1]
    wrow_ref = rest[2 * STREAMS + 2]
    out_ref = rest[2 * STREAMS + 3]

    e = pl.program_id(0)
    fc = pl.program_id(1)
    tile_lo = meta_ref[0, e]
    tile_hi = meta_ref[1, e]
    seg_lo = meta_ref[2, e]
    seg_hi = meta_ref[3, e]

    @pl.when((e == 0) & (fc == 0))
    def _():
        out_ref[...] = jnp.zeros_like(out_ref)

    b2_scale = jnp.where(fc == 0, 1.0, 0.0)
    fch = b1_ref.shape[-1] // STREAMS

    def tile_step(j, _):
        rows = pl.ds(j * TILE, TILE)
        x = x_ref[rows, :].astype(jnp.bfloat16)
        y = b2_scale * b2_ref[0]
        for k in range(STREAMS):
            hk = jnp.dot(x, w1_refs[k][0].astype(jnp.bfloat16),
                         preferred_element_type=jnp.float32)
            hk = jax.nn.gelu(hk + b1_ref[0, :, k * fch:(k + 1) * fch])
            y = y + jnp.dot(hk.astype(jnp.bfloat16),
                            w2_refs[k][0].astype(jnp.bfloat16),
                            preferred_element_type=jnp.float32)
        gidx = j * TILE + lax.broadcasted_iota(jnp.int32, (TILE, 1), 0)
        coef = jnp.where((gidx >= seg_lo) & (gidx < seg_hi),
                         wrow_ref[rows, 0:1], 0.0)
        out_ref[rows, :] += y * coef
        return 0

    lax.fori_loop(tile_lo, tile_hi + 1, tile_step, 0)


def _grouped_mlp(meta, x_sorted, W1, b1, W2, b2, w_sorted128):
    t, h = x_sorted.shape
    e, _, f = W1.shape
    hw = w_sorted128.shape[1]
    fchunk = f // NF

    def w1_map(k):
        return lambda ei, fc, m: (ei, 0, STREAMS * fc + k)

    def w2_map(k):
        return lambda ei, fc, m: (ei, STREAMS * fc + k, 0)

    in_specs = [pl.BlockSpec((t, h), lambda ei, fc, m: (0, 0))]
    in_specs += [pl.BlockSpec((1, h, fchunk), w1_map(k))
                 for k in range(STREAMS)]
    in_specs += [pl.BlockSpec((1, 1, STREAMS * fchunk),
                              lambda ei, fc, m: (ei, 0, fc))]
    in_specs += [pl.BlockSpec((1, fchunk, h), w2_map(k))
                 for k in range(STREAMS)]
    in_specs += [
        pl.BlockSpec((1, 1, h), lambda ei, fc, m: (ei, 0, 0)),
        pl.BlockSpec((t, hw), lambda ei, fc, m: (0, 0)),
    ]
    grid_spec = pltpu.PrefetchScalarGridSpec(
        num_scalar_prefetch=1,
        grid=(e, NF // STREAMS),
        in_specs=in_specs,
        out_specs=pl.BlockSpec((t, h), lambda ei, fc, m: (0, 0)),
    )
    args = ([x_sorted] + [W1] * STREAMS + [b1.reshape(e, 1, f)]
            + [W2] * STREAMS + [b2.reshape(e, 1, h), w_sorted128])
    return pl.pallas_call(
        _moe_body,
        grid_spec=grid_spec,
        out_shape=jax.ShapeDtypeStruct((t, h), jnp.float32),
    )(meta, *args)


def _routing(eid, n_experts):
    t = eid.shape[0]
    i32 = jnp.int32
    oh = (eid[:, None] == jnp.arange(n_experts, dtype=i32)[None, :])
    oh = oh.astype(i32)
    counts = oh.sum(axis=0)
    offsets = jnp.concatenate(
        [jnp.zeros((1,), i32), jnp.cumsum(counts)[:-1].astype(i32)])
    csum = jnp.cumsum(oh, axis=0) - oh
    rank = ((offsets[None, :] + csum) * oh).sum(axis=1).astype(i32)

    seg_lo = offsets
    seg_hi = offsets + counts
    tile_lo = jnp.where(counts > 0, offsets // TILE, 1).astype(i32)
    tile_hi = jnp.where(counts > 0, (seg_hi - 1) // TILE, 0)
    meta = jnp.stack(
        [tile_lo, tile_hi.astype(i32), seg_lo.astype(i32),
         seg_hi.astype(i32)])
    return meta, rank


def kernel(hidden_states, gate_w, alpha, W1, b1, W2, b2):
    t = hidden_states.shape[0]
    e = gate_w.shape[1]

    eid, wcomb = _gate(hidden_states, gate_w, alpha)
    meta, rank = _routing(eid, e)
    wpad = jnp.concatenate(
        [wcomb[:, None], jnp.zeros((t, 127), jnp.float32)], axis=1)
    x_sorted, w_sorted128 = _sc_dispatch(hidden_states, wpad, rank)
    y_sorted = _grouped_mlp(meta, x_sorted, W1, b1, W2, b2, w_sorted128)
    return _sc_gather_rows(y_sorted, rank)
